# Initial kernel scaffold; baseline (speedup 1.0000x reference)
import jax
import jax.numpy as jnp
from jax.experimental import pallas as pl


def kernel(x_drug, x_cmpd, edge_index_drug, edge_index_cmpd, batch_drug, batch_cmpd, W_in_d, W_msg_d, W_upd_d, W_in_c, W_msg_c, W_upd_c, W_q, W1, b1, W2, b2):
    raise NotImplementedError("write your pallas kernel here")


# speedup vs baseline: 2.8502x; 2.8502x over previous
import functools

import jax
import jax.numpy as jnp
from jax import lax
from jax.experimental import pallas as pl
from jax.experimental.pallas import tpu as pltpu
from jax.experimental.pallas import tpu_sc as plsc

N = 10000
E = 160000
D = 256
H = 256
FFN_H = 512
OUT = 1
G = 128
STEPS = 3

HALF = H // 2
NC, NS = 2, 16
KE = 80
EPT = E // NS
NIT = EPT // KE
RPT = 632
NPAD = RPT * NS

BR = 400
GRID = N // BR


@functools.cache
def _sc_mesh():
    return plsc.VectorSubcoreMesh(
        core_axis_name="c", subcore_axis_name="s",
        num_cores=NC, num_subcores=NS)


def _agg_body(h2, srcall, dst, zeros, out, idx_v, dsti_v, rows_v, acc, sem):
    c = lax.axis_index("c")
    s = lax.axis_index("s")
    row0 = s * RPT
    pltpu.sync_copy(zeros, acc.at[pl.ds(row0, RPT)])
    plsc.subcore_barrier()
    ebase = c * E + s * EPT
    dbase = s * EPT

    def body(i, carry):
        off = i * KE
        pltpu.sync_copy(srcall.at[pl.ds(ebase + off, KE)], idx_v)
        pltpu.sync_copy(dst.at[pl.ds(dbase + off, KE)], dsti_v)
        pltpu.async_copy(h2.at[idx_v], rows_v, sem).wait()
        pltpu.sync_copy(rows_v, acc.at[dsti_v], add=True)
        return carry

    lax.fori_loop(0, NIT, body, 0)
    plsc.subcore_barrier()
    pltpu.sync_copy(acc.at[pl.ds(row0, RPT)],
                    out.at[pl.ds(c * NPAD + row0, RPT)])


@functools.cache
def _agg_kernel():
    return pl.kernel(
        _agg_body,
        out_type=jax.ShapeDtypeStruct((2 * NPAD, HALF), jnp.float32),
        mesh=_sc_mesh(),
        scratch_types=[
            pltpu.VMEM((KE,), jnp.int32),
            pltpu.VMEM((KE,), jnp.int32),
            pltpu.VMEM((KE, HALF), jnp.float32),
            pltpu.VMEM_SHARED((NPAD, HALF), jnp.float32),
            pltpu.SemaphoreType.DMA,
        ],
    )


def _agg(h2, srcall, dst, zeros):
    return _agg_kernel()(h2, srcall, dst, zeros)


def _proj_body(x_ref, w_ref, o_ref):
    h = jnp.maximum(
        jnp.dot(x_ref[...], w_ref[...], preferred_element_type=jnp.float32), 0.0)
    o_ref[0] = h[:, :HALF]
    o_ref[1] = h[:, HALF:]


def _proj(x, w):
    return pl.pallas_call(
        _proj_body,
        grid=(GRID,),
        in_specs=[
            pl.BlockSpec((BR, D), lambda i: (i, 0)),
            pl.BlockSpec((D, H), lambda i: (0, 0)),
        ],
        out_specs=pl.BlockSpec((2, BR, HALF), lambda i: (0, i, 0)),
        out_shape=jax.ShapeDtypeStruct((2, N, HALF), jnp.float32),
    )(x, w)


def _wmu_body(wm_ref, wu_ref, o_ref):
    o_ref[...] = jnp.dot(
        wm_ref[...], wu_ref[...], preferred_element_type=jnp.float32)


def _wmu(wm, wu):
    return pl.pallas_call(
        _wmu_body,
        out_shape=jax.ShapeDtypeStruct((H, H), jnp.float32),
    )(wm, wu)


def _update_body(a_ref, h0_ref, wmu_ref, o_ref):
    a = jnp.concatenate([a_ref[0], a_ref[1]], axis=1)
    h0 = jnp.concatenate([h0_ref[0], h0_ref[1]], axis=1)
    h = jnp.maximum(
        h0 + jnp.dot(a, wmu_ref[...], preferred_element_type=jnp.float32), 0.0)
    o_ref[0] = h[:, :HALF]
    o_ref[1] = h[:, HALF:]


def _update(a3, h03, wmu):
    return pl.pallas_call(
        _update_body,
        grid=(GRID,),
        in_specs=[
            pl.BlockSpec((2, BR, HALF), lambda i: (0, i, 0)),
            pl.BlockSpec((2, BR, HALF), lambda i: (0, i, 0)),
            pl.BlockSpec((H, H), lambda i: (0, 0)),
        ],
        out_specs=pl.BlockSpec((2, BR, HALF), lambda i: (0, i, 0)),
        out_shape=jax.ShapeDtypeStruct((2, N, HALF), jnp.float32),
    )(a3, h03, wmu)


def _drug_readout_body(h_ref, b_ref, wq_ref, q_ref):
    h = jnp.concatenate([h_ref[0], h_ref[1]], axis=1)
    oh = (b_ref[...] == lax.broadcasted_iota(jnp.int32, (N, G), 1)
          ).astype(jnp.float32)
    sums = lax.dot_general(oh, h, (((0,), (0,)), ((), ())),
                           preferred_element_type=jnp.float32)
    cnt = lax.dot_general(oh, jnp.ones((N, 1), jnp.float32),
                          (((0,), (0,)), ((), ())),
                          preferred_element_type=jnp.float32)
    emb = sums / jnp.maximum(cnt, 1.0)
    q_ref[...] = jnp.dot(emb, wq_ref[...],
                         preferred_element_type=jnp.float32)


def _drug_readout(h3, b2d, wq):
    return pl.pallas_call(
        _drug_readout_body,
        out_shape=jax.ShapeDtypeStruct((G, H), jnp.float32),
    )(h3, b2d, wq)


def _cmpd_readout_body(h_ref, b_ref, q_ref, w1_ref, b1_ref, w2_ref, b2_ref,
                       out_ref, ent_ref):
    h = jnp.concatenate([h_ref[0], h_ref[1]], axis=1)
    mask = b_ref[...] == lax.broadcasted_iota(jnp.int32, (N, G), 1)
    oh = mask.astype(jnp.float32)
    qb = jnp.dot(oh, q_ref[...], preferred_element_type=jnp.float32)
    scores = jnp.sum(h * qb, axis=1, keepdims=True)
    masked = jnp.where(mask, scores, -1e30)
    smax = jnp.max(masked, axis=0, keepdims=True)
    srow = jnp.sum(oh * smax, axis=1, keepdims=True)
    p = jnp.exp(scores - srow)
    denom = lax.dot_general(oh, p, (((0,), (0,)), ((), ())),
                            preferred_element_type=jnp.float32)
    attn = p / jnp.dot(oh, denom, preferred_element_type=jnp.float32)
    emb = lax.dot_general(oh, attn * h, (((0,), (0,)), ((), ())),
                          preferred_element_type=jnp.float32)
    plogp = attn * jnp.log(attn + 1e-12)
    ent = -lax.dot_general(oh, plogp, (((0,), (0,)), ((), ())),
                           preferred_element_type=jnp.float32)
    hid = jnp.maximum(
        jnp.dot(emb, w1_ref[...], preferred_element_type=jnp.float32)
        + b1_ref[...], 0.0)
    out = (jnp.dot(hid, w2_ref[...], preferred_element_type=jnp.float32)
           + b2_ref[...])
    out_ref[...] = jax.nn.sigmoid(out)
    ent_ref[...] = ent


def _cmpd_readout(h3, b2d, q, w1, b1, w2, b2):
    return pl.pallas_call(
        _cmpd_readout_body,
        out_shape=(
            jax.ShapeDtypeStruct((G, OUT), jnp.float32),
            jax.ShapeDtypeStruct((G, 1), jnp.float32),
        ),
    )(h3, b2d, q, w1, b1, w2, b2)


def _encoder(x, src, dst, w_in, wmu, zeros):
    h03 = _proj(x, w_in)
    srcall = jnp.concatenate([src, src + N])
    h3 = h03
    for _ in range(STEPS):
        a2 = _agg(h3.reshape(2 * N, HALF), srcall, dst, zeros)
        h3 = _update(a2.reshape(2, NPAD, HALF), h03, wmu)
    return h3


def kernel(x_drug, x_cmpd, edge_index_drug, edge_index_cmpd, batch_drug,
           batch_cmpd, W_in_d, W_msg_d, W_upd_d, W_in_c, W_msg_c, W_upd_c,
           W_q, W1, b1, W2, b2):
    zeros = jnp.zeros((RPT, HALF), jnp.float32)
    wmu_d = _wmu(W_msg_d, W_upd_d)
    wmu_c = _wmu(W_msg_c, W_upd_c)
    h_d = _encoder(x_drug, edge_index_drug[0], edge_index_drug[1],
                   W_in_d, wmu_d, zeros)
    h_c = _encoder(x_cmpd, edge_index_cmpd[0], edge_index_cmpd[1],
                   W_in_c, wmu_c, zeros)
    q = _drug_readout(h_d, batch_drug.reshape(N, 1), W_q)
    out, ent = _cmpd_readout(h_c, batch_cmpd.reshape(N, 1), q,
                             W1, b1.reshape(1, FFN_H), W2, b2.reshape(1, OUT))
    return (out, ent[:, 0])
